# baseline (speedup 1.0000x reference)
import functools

import jax
import jax.numpy as jnp
from jax import lax
from jax.experimental import pallas as pl
from jax.experimental.pallas import tpu as pltpu
from jax.experimental.pallas import tpu_sc as plsc

_NC = 2
_NS = 16
_L = 16

_BUF_WORDS = 65536
_UNROLL = 8


@functools.lru_cache(maxsize=None)
def _sc_zero_fill(n_words):
    n_workers = _NC * _NS
    per_worker = n_words // n_workers
    assert per_worker * n_workers == n_words
    n_copies = per_worker // _BUF_WORDS
    assert n_copies * _BUF_WORDS == per_worker

    mesh = plsc.VectorSubcoreMesh(
        core_axis_name="c", subcore_axis_name="s",
        num_cores=_NC, num_subcores=_NS,
    )

    @functools.partial(
        pl.kernel,
        out_type=jax.ShapeDtypeStruct((n_words,), jnp.float32),
        mesh=mesh,
        scratch_types=[
            pltpu.VMEM((_BUF_WORDS,), jnp.float32),
            pltpu.SemaphoreType.DMA,
        ],
    )
    def sc_zero(out_hbm, zbuf, sem):
        wid = lax.axis_index("s") * _NC + lax.axis_index("c")
        zeros = jnp.zeros((_L,), jnp.float32)

        def zero_body(i, carry):
            base = i * (_L * _UNROLL)
            for u in range(_UNROLL):
                zbuf[pl.ds(base + u * _L, _L)] = zeros
            return carry

        lax.fori_loop(0, _BUF_WORDS // (_L * _UNROLL), zero_body, 0)

        base = wid * per_worker
        copies = [
            pltpu.async_copy(
                zbuf,
                out_hbm.at[pl.ds(base + k * _BUF_WORDS, _BUF_WORDS)],
                sem,
            )
            for k in range(n_copies)
        ]
        for c in copies:
            c.wait()

    return sc_zero


def kernel(x):
    B, C = x.shape
    out = _sc_zero_fill(B * C)()
    return out.reshape(B, C)
